# baseline (speedup 1.0000x reference)
import jax
import jax.numpy as jnp
from jax import lax
from jax.experimental import pallas as pl
from jax.experimental.pallas import tpu as pltpu
import jax.experimental.pallas.tpu_sc as plsc

N_NODES = 10000
N_EDGES = 640000
N_GRAPHS = 64
EPS = 1e-5
F_IN = 16
F_HID = 256
F_MLP = 512

NC = 2
NS = 16
CHUNK = 80
ROWS = N_EDGES // CHUNK
STAGE = 50
K = 5
STAGE_A = 125
STAGE_B = 50
BLK = 1000
GRID = N_NODES // BLK

_f32 = jnp.float32


def _mesh():
    return plsc.VectorSubcoreMesh(core_axis_name="c", subcore_axis_name="s")


_SC_PARAMS = pltpu.CompilerParams(use_tc_tiling_on_sc=False)


def _deg_body(dst_hbm, out_hbm, zbuf, ones_v, idx_v, shared, sem):
    c = lax.axis_index("c")
    s = lax.axis_index("s")

    @pl.when(s == 0)
    def _init():
        def fz(i, carry):
            zbuf[pl.ds(i * 16, 16)] = jnp.zeros((16,), _f32)
            return carry
        lax.fori_loop(0, zbuf.shape[0] // 16, fz, 0)
        for g in range(N_NODES // zbuf.shape[0]):
            pltpu.sync_copy(zbuf, shared.at[pl.ds(g * zbuf.shape[0], zbuf.shape[0])])

    for i in range(CHUNK // 16):
        ones_v[pl.ds(i * 16, 16)] = jnp.ones((16,), _f32)
    plsc.subcore_barrier()

    rows_w = ROWS // (NC * NS)
    base = (c * NS + s) * rows_w

    def stage_body(t, carry):
        pltpu.sync_copy(dst_hbm.at[pl.ds(base + t * STAGE, STAGE)], idx_v)

        def grp(g, carry2):
            hs = []
            for b in range(K):
                j = g * K + b
                hs.append(pltpu.async_copy(ones_v, shared.at[idx_v.at[j]], sem,
                                           add=True))
            for h in hs:
                h.wait()
            return carry2
        lax.fori_loop(0, STAGE // K, grp, 0)
        return carry
    lax.fori_loop(0, rows_w // STAGE, stage_body, 0)

    plsc.subcore_barrier()

    @pl.when(s == 0)
    def _out():
        pltpu.sync_copy(shared, out_hbm.at[c])


def _sc_deg(dst2d):
    return pl.kernel(
        _deg_body,
        out_type=jax.ShapeDtypeStruct((NC, N_NODES), _f32),
        mesh=_mesh(),
        compiler_params=_SC_PARAMS,
        scratch_types=[
            pltpu.VMEM((2000,), _f32),
            pltpu.VMEM((CHUNK,), _f32),
            pltpu.VMEM((STAGE, CHUNK), jnp.int32),
            pltpu.VMEM_SHARED((N_NODES,), _f32),
            pltpu.SemaphoreType.DMA,
        ],
    )(dst2d)


def _agg_worker(src_hbm, dst_hbm, table_hbm, shared, idx_s, idx_d, rows,
                semg, sems, base, nrows):
    stage = idx_s.shape[0]
    nq = (stage - 1) // 4
    npipe = nq * 4

    def gather(j, b):
        return pltpu.async_copy(table_hbm.at[idx_s.at[j]], rows.at[b],
                                semg[b])

    def gather_wait(j, b):
        pltpu.make_async_copy(table_hbm.at[idx_s.at[j]], rows.at[b],
                              semg[b]).wait()

    def scatter(j, b):
        return pltpu.async_copy(rows.at[b], shared.at[idx_d.at[j]], sems[b],
                                add=True)

    def scatter_wait(j, b):
        pltpu.make_async_copy(rows.at[b], shared.at[idx_d.at[j]],
                              sems[b]).wait()

    def stage_body(t, carry):
        r0 = base + t * stage
        pltpu.sync_copy(src_hbm.at[pl.ds(r0, stage)], idx_s)
        pltpu.sync_copy(dst_hbm.at[pl.ds(r0, stage)], idx_d)
        gather(0, 0)
        gather(1, 1)

        def quad(q, carry2):
            for b in range(4):
                j = q * 4 + b
                b2 = (b + 2) % 4

                @pl.when(jnp.logical_and(j >= 2, j + 2 < npipe))
                def _free_buf():
                    scatter_wait(j - 2, b2)

                @pl.when(j + 2 < npipe)
                def _prefetch():
                    gather(j + 2, b2)
                gather_wait(j, b)
                scatter(j, b)
            return carry2
        lax.fori_loop(0, nq, quad, 0)
        for b in range(4):
            scatter_wait(npipe - 4 + b, b)
        for w in range(npipe, stage):
            gather(w, 0).wait()
            scatter(w, 0).wait()
        return carry
    lax.fori_loop(0, nrows // stage, stage_body, 0)


def _agg16_body(src_hbm, dst_hbm, table_hbm, out_hbm, idx_s, idx_d, rows,
                shared, sg0, sg1, sg2, sg3, ss0, ss1, ss2, ss3):
    c = lax.axis_index("c")
    s = lax.axis_index("s")

    @pl.when(s == 0)
    def _init():
        pltpu.sync_copy(table_hbm, shared)
    plsc.subcore_barrier()

    rows_w = ROWS // (NC * NS)
    _agg_worker(src_hbm, dst_hbm, table_hbm, shared, idx_s, idx_d, rows,
                (sg0, sg1, sg2, sg3), (ss0, ss1, ss2, ss3),
                (c * NS + s) * rows_w, rows_w)

    plsc.subcore_barrier()

    @pl.when(s == 0)
    def _out():
        pltpu.sync_copy(shared, out_hbm.at[c])


def _sc_agg16(src2d, dst2d, xs):
    return pl.kernel(
        _agg16_body,
        out_type=jax.ShapeDtypeStruct((NC, N_NODES, F_IN), _f32),
        mesh=_mesh(),
        compiler_params=_SC_PARAMS,
        scratch_types=[
            pltpu.VMEM((STAGE_A, CHUNK), jnp.int32),
            pltpu.VMEM((STAGE_A, CHUNK), jnp.int32),
            pltpu.VMEM((4, CHUNK, F_IN), _f32),
            pltpu.VMEM_SHARED((N_NODES, F_IN), _f32),
        ] + [pltpu.SemaphoreType.DMA] * 8,
    )(src2d, dst2d, xs)


def _agg256_body(src_hbm, dst_hbm, t0_hbm, t1_hbm, out_hbm, idx_s, idx_d,
                 rows, shared, sg0, sg1, sg2, sg3, ss0, ss1, ss2, ss3):
    c = lax.axis_index("c")
    s = lax.axis_index("s")

    def run(table_hbm):
        @pl.when(s == 0)
        def _init():
            pltpu.sync_copy(table_hbm, shared)
        plsc.subcore_barrier()

        rows_w = ROWS // NS
        _agg_worker(src_hbm, dst_hbm, table_hbm, shared, idx_s, idx_d, rows,
                    (sg0, sg1, sg2, sg3), (ss0, ss1, ss2, ss3),
                    s * rows_w, rows_w)

        plsc.subcore_barrier()

        @pl.when(s == 0)
        def _out():
            pltpu.sync_copy(shared, out_hbm.at[c])

    @pl.when(c == 0)
    def _c0():
        run(t0_hbm)

    @pl.when(c == 1)
    def _c1():
        run(t1_hbm)


def _sc_agg256(src2d, dst2d, hs0, hs1):
    half = F_HID // 2
    return pl.kernel(
        _agg256_body,
        out_type=jax.ShapeDtypeStruct((NC, N_NODES, half), _f32),
        mesh=_mesh(),
        compiler_params=_SC_PARAMS,
        scratch_types=[
            pltpu.VMEM((STAGE_B, CHUNK), jnp.int32),
            pltpu.VMEM((STAGE_B, CHUNK), jnp.int32),
            pltpu.VMEM((4, CHUNK, half), _f32),
            pltpu.VMEM_SHARED((N_NODES, half), _f32),
        ] + [pltpu.SemaphoreType.DMA] * 8,
    )(src2d, dst2d, hs0, hs1)


def _prep_body(degp_ref, x_ref, dinv_ref, xs_ref):
    deg = degp_ref[:, 0:1] + degp_ref[:, 1:2] + 1.0
    dinv = lax.rsqrt(deg)
    dinv_ref[:, :] = dinv
    xs_ref[:, :] = x_ref[:, :] * dinv


def _tc_prep(degp_t, tree_x):
    return pl.pallas_call(
        _prep_body,
        out_shape=[
            jax.ShapeDtypeStruct((N_NODES, 1), _f32),
            jax.ShapeDtypeStruct((N_NODES, F_IN), _f32),
        ],
    )(degp_t, tree_x)


def _ln_relu(pre, g, be):
    m = jnp.mean(pre, axis=-1, keepdims=True)
    v = jnp.mean((pre - m) ** 2, axis=-1, keepdims=True)
    return jnp.maximum((pre - m) / jnp.sqrt(v + EPS) * g + be, 0.0)


def _l1_body(p0, p1, xs, dinv, W, b, g, be, hs0, hs1):
    dv = dinv[:, :]
    t = (p0[:, :] + p1[:, :] - xs[:, :]) * dv
    pre = jnp.dot(t, W[:, :], preferred_element_type=_f32, precision=lax.Precision.HIGHEST) + b[:, :]
    h = _ln_relu(pre, g[:, :], be[:, :])
    hsv = h * dv
    half = F_HID // 2
    hs0[:, :] = hsv[:, :half]
    hs1[:, :] = hsv[:, half:]


def _tc_layer1(p0, p1, xs, dinv, W_g1, b_g1, g_n1, be_n1):
    half = F_HID // 2
    blk_r = lambda i: (i, 0)
    blk_w = lambda i: (0, 0)
    return pl.pallas_call(
        _l1_body,
        grid=(GRID,),
        in_specs=[
            pl.BlockSpec((BLK, F_IN), blk_r),
            pl.BlockSpec((BLK, F_IN), blk_r),
            pl.BlockSpec((BLK, F_IN), blk_r),
            pl.BlockSpec((BLK, 1), blk_r),
            pl.BlockSpec((F_IN, F_HID), blk_w),
            pl.BlockSpec((1, F_HID), blk_w),
            pl.BlockSpec((1, F_HID), blk_w),
            pl.BlockSpec((1, F_HID), blk_w),
        ],
        out_specs=[
            pl.BlockSpec((BLK, half), blk_r),
            pl.BlockSpec((BLK, half), blk_r),
        ],
        out_shape=[
            jax.ShapeDtypeStruct((N_NODES, half), _f32),
            jax.ShapeDtypeStruct((N_NODES, half), _f32),
        ],
    )(p0, p1, xs, dinv, W_g1, b_g1, g_n1, be_n1)


def _head_body(a0, a1, dinv, bat, W2a, W2b, b2, g2, be2,
               Wf1, bf1, gf1, bef1, Wf2, bf2, gf2, bef2,
               Wf3, bf3, gf3, bef3, wo, bo, out, acc):
    i = pl.program_id(0)

    @pl.when(i == 0)
    def _():
        acc[:, :] = jnp.full((N_GRAPHS, F_HID), -jnp.inf, _f32)

    dv = dinv[:, :]
    pre = (jnp.dot(a0[:, :] * dv, W2a[:, :], preferred_element_type=_f32, precision=lax.Precision.HIGHEST)
           + jnp.dot(a1[:, :] * dv, W2b[:, :], preferred_element_type=_f32, precision=lax.Precision.HIGHEST)
           + b2[:, :])
    h2 = _ln_relu(pre, g2[:, :], be2[:, :])

    ids = bat[:, :]
    QR = BLK // 4
    for q in range(4):
        ids_q = ids[q * QR:(q + 1) * QR, :]
        h2_q = h2[q * QR:(q + 1) * QR, :]
        lo = bat[q * QR, 0]
        hi = bat[(q + 1) * QR - 1, 0]

        def seg(gid, carry, ids_q=ids_q, h2_q=h2_q):
            mm = jnp.max(jnp.where(ids_q == gid, h2_q, -jnp.inf), axis=0,
                         keepdims=True)
            acc[pl.ds(gid, 1), :] = jnp.maximum(acc[pl.ds(gid, 1), :], mm)
            return carry
        lax.fori_loop(lo, hi + 1, seg, 0)

    @pl.when(i == pl.num_programs(0) - 1)
    def _mlp():
        z = acc[:, :]
        z = _ln_relu(jnp.dot(z, Wf1[:, :], preferred_element_type=_f32, precision=lax.Precision.HIGHEST)
                     + bf1[:, :], gf1[:, :], bef1[:, :])
        z = _ln_relu(jnp.dot(z, Wf2[:, :], preferred_element_type=_f32, precision=lax.Precision.HIGHEST)
                     + bf2[:, :], gf2[:, :], bef2[:, :])
        z = _ln_relu(jnp.dot(z, Wf3[:, :], preferred_element_type=_f32, precision=lax.Precision.HIGHEST)
                     + bf3[:, :], gf3[:, :], bef3[:, :])
        out[:, :] = jnp.sum(z * wo[:, :], axis=1, keepdims=True) + bo[:, :]


def _tc_head(a0, a1, dinv, bat2d, W2a, W2b, b2, g2, be2,
             Wf1, bf1, gf1, bef1, Wf2, bf2, gf2, bef2,
             Wf3, bf3, gf3, bef3, wo, bo):
    half = F_HID // 2
    blk_r = lambda i: (i, 0)
    blk_w = lambda i: (0, 0)
    vec = lambda d: pl.BlockSpec((1, d), blk_w)
    return pl.pallas_call(
        _head_body,
        grid=(GRID,),
        in_specs=[
            pl.BlockSpec((BLK, half), blk_r),
            pl.BlockSpec((BLK, half), blk_r),
            pl.BlockSpec((BLK, 1), blk_r),
            pl.BlockSpec((BLK, 1), blk_r),
            pl.BlockSpec((half, F_HID), blk_w),
            pl.BlockSpec((half, F_HID), blk_w),
            vec(F_HID), vec(F_HID), vec(F_HID),
            pl.BlockSpec((F_HID, F_MLP), blk_w),
            vec(F_MLP), vec(F_MLP), vec(F_MLP),
            pl.BlockSpec((F_MLP, F_MLP), blk_w),
            vec(F_MLP), vec(F_MLP), vec(F_MLP),
            pl.BlockSpec((F_MLP, F_MLP), blk_w),
            vec(F_MLP), vec(F_MLP), vec(F_MLP),
            vec(F_MLP), vec(1),
        ],
        out_specs=pl.BlockSpec((N_GRAPHS, 1), blk_w),
        out_shape=jax.ShapeDtypeStruct((N_GRAPHS, 1), _f32),
        scratch_shapes=[pltpu.VMEM((N_GRAPHS, F_HID), _f32)],
    )(a0, a1, dinv, bat2d, W2a, W2b, b2, g2, be2,
      Wf1, bf1, gf1, bef1, Wf2, bf2, gf2, bef2,
      Wf3, bf3, gf3, bef3, wo, bo)


def kernel(tree_x, edge_index, batch,
           W_g1, b_g1, g_n1, be_n1,
           W_g2, b_g2, g_n2, be_n2,
           W_f1, b_f1, g_f1, be_f1,
           W_f2, b_f2, g_f2, be_f2,
           W_f3, b_f3, g_f3, be_f3,
           W_o, b_o):
    half = F_HID // 2
    src2d = edge_index[0].reshape(ROWS, CHUNK)
    dst2d = edge_index[1].reshape(ROWS, CHUNK)

    degp = _sc_deg(dst2d)
    dinv, xs = _tc_prep(degp.T, tree_x)
    aggp = _sc_agg16(src2d, dst2d, xs)
    hs0, hs1 = _tc_layer1(aggp[0], aggp[1], xs, dinv, W_g1,
                          b_g1.reshape(1, F_HID), g_n1.reshape(1, F_HID),
                          be_n1.reshape(1, F_HID))
    ap = _sc_agg256(src2d, dst2d, hs0, hs1)
    return _tc_head(ap[0], ap[1], dinv, batch.reshape(N_NODES, 1),
                    W_g2[:half], W_g2[half:],
                    b_g2.reshape(1, F_HID), g_n2.reshape(1, F_HID),
                    be_n2.reshape(1, F_HID),
                    W_f1, b_f1.reshape(1, F_MLP), g_f1.reshape(1, F_MLP),
                    be_f1.reshape(1, F_MLP),
                    W_f2, b_f2.reshape(1, F_MLP), g_f2.reshape(1, F_MLP),
                    be_f2.reshape(1, F_MLP),
                    W_f3, b_f3.reshape(1, F_MLP), g_f3.reshape(1, F_MLP),
                    be_f3.reshape(1, F_MLP),
                    W_o.reshape(1, F_MLP), b_o.reshape(1, 1))


# speedup vs baseline: 1.0376x; 1.0376x over previous
import jax
import jax.numpy as jnp
from jax import lax
from jax.experimental import pallas as pl
from jax.experimental.pallas import tpu as pltpu
import jax.experimental.pallas.tpu_sc as plsc

N_NODES = 10000
N_EDGES = 640000
N_GRAPHS = 64
EPS = 1e-5
F_IN = 16
F_HID = 256
F_MLP = 512

NC = 2
NS = 16
CHUNK = 80
ROWS = N_EDGES // CHUNK
STAGE = 50
K = 5
STAGE_A = 125
CHUNK2 = 64
ROWS2 = N_EDGES // CHUNK2
BLK = 1000
GRID = N_NODES // BLK

_f32 = jnp.float32


def _mesh():
    return plsc.VectorSubcoreMesh(core_axis_name="c", subcore_axis_name="s")


_SC_PARAMS = pltpu.CompilerParams(use_tc_tiling_on_sc=False)


def _deg_body(dst_hbm, out_hbm, zbuf, ones_v, idx_v, shared, sem):
    c = lax.axis_index("c")
    s = lax.axis_index("s")

    @pl.when(s == 0)
    def _init():
        def fz(i, carry):
            zbuf[pl.ds(i * 16, 16)] = jnp.zeros((16,), _f32)
            return carry
        lax.fori_loop(0, zbuf.shape[0] // 16, fz, 0)
        for g in range(N_NODES // zbuf.shape[0]):
            pltpu.sync_copy(zbuf, shared.at[pl.ds(g * zbuf.shape[0], zbuf.shape[0])])

    for i in range(CHUNK // 16):
        ones_v[pl.ds(i * 16, 16)] = jnp.ones((16,), _f32)
    plsc.subcore_barrier()

    rows_w = ROWS // (NC * NS)
    base = (c * NS + s) * rows_w

    def stage_body(t, carry):
        pltpu.sync_copy(dst_hbm.at[pl.ds(base + t * STAGE, STAGE)], idx_v)

        def grp(g, carry2):
            hs = []
            for b in range(K):
                j = g * K + b
                hs.append(pltpu.async_copy(ones_v, shared.at[idx_v.at[j]], sem,
                                           add=True))
            for h in hs:
                h.wait()
            return carry2
        lax.fori_loop(0, STAGE // K, grp, 0)
        return carry
    lax.fori_loop(0, rows_w // STAGE, stage_body, 0)

    plsc.subcore_barrier()

    @pl.when(s == 0)
    def _out():
        pltpu.sync_copy(shared, out_hbm.at[c])


def _sc_deg(dst2d):
    return pl.kernel(
        _deg_body,
        out_type=jax.ShapeDtypeStruct((NC, N_NODES), _f32),
        mesh=_mesh(),
        compiler_params=_SC_PARAMS,
        scratch_types=[
            pltpu.VMEM((2000,), _f32),
            pltpu.VMEM((CHUNK,), _f32),
            pltpu.VMEM((STAGE, CHUNK), jnp.int32),
            pltpu.VMEM_SHARED((N_NODES,), _f32),
            pltpu.SemaphoreType.DMA,
        ],
    )(dst2d)


def _agg_worker(src_hbm, dst_hbm, table_hbm, shared, idx_s, idx_d, rows,
                semg, sems, base, nrows):
    stage = idx_s.shape[0]
    nq = (stage - 1) // 4
    npipe = nq * 4

    def gather(j, b):
        return pltpu.async_copy(table_hbm.at[idx_s.at[j]], rows.at[b],
                                semg[b])

    def gather_wait(j, b):
        pltpu.make_async_copy(table_hbm.at[idx_s.at[j]], rows.at[b],
                              semg[b]).wait()

    def scatter(j, b):
        return pltpu.async_copy(rows.at[b], shared.at[idx_d.at[j]], sems[b],
                                add=True)

    def scatter_wait(j, b):
        pltpu.make_async_copy(rows.at[b], shared.at[idx_d.at[j]],
                              sems[b]).wait()

    def stage_body(t, carry):
        r0 = base + t * stage
        pltpu.sync_copy(src_hbm.at[pl.ds(r0, stage)], idx_s)
        pltpu.sync_copy(dst_hbm.at[pl.ds(r0, stage)], idx_d)
        gather(0, 0)
        gather(1, 1)

        def quad(q, carry2):
            for b in range(4):
                j = q * 4 + b
                b2 = (b + 2) % 4

                @pl.when(jnp.logical_and(j >= 2, j + 2 < npipe))
                def _free_buf():
                    scatter_wait(j - 2, b2)

                @pl.when(j + 2 < npipe)
                def _prefetch():
                    gather(j + 2, b2)
                gather_wait(j, b)
                scatter(j, b)
            return carry2
        lax.fori_loop(0, nq, quad, 0)
        for b in range(4):
            scatter_wait(npipe - 4 + b, b)
        for w in range(npipe, stage):
            gather(w, 0).wait()
            scatter(w, 0).wait()
        return carry
    lax.fori_loop(0, nrows // stage, stage_body, 0)


def _agg16_body(src_hbm, dst_hbm, table_hbm, out_hbm, idx_s, idx_d, rows,
                shared, sg0, sg1, sg2, sg3, ss0, ss1, ss2, ss3):
    c = lax.axis_index("c")
    s = lax.axis_index("s")

    @pl.when(s == 0)
    def _init():
        pltpu.sync_copy(table_hbm, shared)
    plsc.subcore_barrier()

    rows_w = ROWS // (NC * NS)
    _agg_worker(src_hbm, dst_hbm, table_hbm, shared, idx_s, idx_d, rows,
                (sg0, sg1, sg2, sg3), (ss0, ss1, ss2, ss3),
                (c * NS + s) * rows_w, rows_w)

    plsc.subcore_barrier()

    @pl.when(s == 0)
    def _out():
        pltpu.sync_copy(shared, out_hbm.at[c])


def _sc_agg16(src2d, dst2d, xs):
    return pl.kernel(
        _agg16_body,
        out_type=jax.ShapeDtypeStruct((NC, N_NODES, F_IN), _f32),
        mesh=_mesh(),
        compiler_params=_SC_PARAMS,
        scratch_types=[
            pltpu.VMEM((STAGE_A, CHUNK), jnp.int32),
            pltpu.VMEM((STAGE_A, CHUNK), jnp.int32),
            pltpu.VMEM((4, CHUNK, F_IN), _f32),
            pltpu.VMEM_SHARED((N_NODES, F_IN), _f32),
        ] + [pltpu.SemaphoreType.DMA] * 8,
    )(src2d, dst2d, xs)


def _agg256_body(src_hbm, dst_hbm, t0_hbm, t1_hbm, out_hbm, idx_s, idx_d,
                 rows, shared, sg0, sg1, sg2, sg3, ss0, ss1, ss2, ss3):
    c = lax.axis_index("c")
    s = lax.axis_index("s")

    def run(table_hbm):
        @pl.when(s == 0)
        def _init():
            pltpu.sync_copy(table_hbm, shared)
        plsc.subcore_barrier()

        rows_w = ROWS2 // NS
        _agg_worker(src_hbm, dst_hbm, table_hbm, shared, idx_s, idx_d, rows,
                    (sg0, sg1, sg2, sg3), (ss0, ss1, ss2, ss3),
                    s * rows_w, rows_w)

        plsc.subcore_barrier()

        @pl.when(s == 0)
        def _out():
            pltpu.sync_copy(shared, out_hbm.at[c])

    @pl.when(c == 0)
    def _c0():
        run(t0_hbm)

    @pl.when(c == 1)
    def _c1():
        run(t1_hbm)


def _sc_agg256(src2d, dst2d, hs0, hs1):
    half = F_HID // 2
    return pl.kernel(
        _agg256_body,
        out_type=jax.ShapeDtypeStruct((NC, N_NODES, half), _f32),
        mesh=_mesh(),
        compiler_params=_SC_PARAMS,
        scratch_types=[
            pltpu.VMEM((STAGE_A, CHUNK2), jnp.int32),
            pltpu.VMEM((STAGE_A, CHUNK2), jnp.int32),
            pltpu.VMEM((4, CHUNK2, half), _f32),
            pltpu.VMEM_SHARED((N_NODES, half), _f32),
        ] + [pltpu.SemaphoreType.DMA] * 8,
    )(src2d, dst2d, hs0, hs1)


def _prep_body(degp_ref, x_ref, dinv_ref, xs_ref):
    deg = degp_ref[:, 0:1] + degp_ref[:, 1:2] + 1.0
    dinv = lax.rsqrt(deg)
    dinv_ref[:, :] = dinv
    xs_ref[:, :] = x_ref[:, :] * dinv


def _tc_prep(degp_t, tree_x):
    return pl.pallas_call(
        _prep_body,
        out_shape=[
            jax.ShapeDtypeStruct((N_NODES, 1), _f32),
            jax.ShapeDtypeStruct((N_NODES, F_IN), _f32),
        ],
    )(degp_t, tree_x)


def _ln_relu(pre, g, be):
    m = jnp.mean(pre, axis=-1, keepdims=True)
    v = jnp.mean((pre - m) ** 2, axis=-1, keepdims=True)
    return jnp.maximum((pre - m) / jnp.sqrt(v + EPS) * g + be, 0.0)


def _l1_body(p0, p1, xs, dinv, W, b, g, be, hs0, hs1):
    dv = dinv[:, :]
    t = (p0[:, :] + p1[:, :] - xs[:, :]) * dv
    pre = jnp.dot(t, W[:, :], preferred_element_type=_f32, precision=lax.Precision.HIGHEST) + b[:, :]
    h = _ln_relu(pre, g[:, :], be[:, :])
    hsv = h * dv
    half = F_HID // 2
    hs0[:, :] = hsv[:, :half]
    hs1[:, :] = hsv[:, half:]


def _tc_layer1(p0, p1, xs, dinv, W_g1, b_g1, g_n1, be_n1):
    half = F_HID // 2
    blk_r = lambda i: (i, 0)
    blk_w = lambda i: (0, 0)
    return pl.pallas_call(
        _l1_body,
        grid=(GRID,),
        in_specs=[
            pl.BlockSpec((BLK, F_IN), blk_r),
            pl.BlockSpec((BLK, F_IN), blk_r),
            pl.BlockSpec((BLK, F_IN), blk_r),
            pl.BlockSpec((BLK, 1), blk_r),
            pl.BlockSpec((F_IN, F_HID), blk_w),
            pl.BlockSpec((1, F_HID), blk_w),
            pl.BlockSpec((1, F_HID), blk_w),
            pl.BlockSpec((1, F_HID), blk_w),
        ],
        out_specs=[
            pl.BlockSpec((BLK, half), blk_r),
            pl.BlockSpec((BLK, half), blk_r),
        ],
        out_shape=[
            jax.ShapeDtypeStruct((N_NODES, half), _f32),
            jax.ShapeDtypeStruct((N_NODES, half), _f32),
        ],
    )(p0, p1, xs, dinv, W_g1, b_g1, g_n1, be_n1)


def _head_body(a0, a1, dinv, bat, W2a, W2b, b2, g2, be2,
               Wf1, bf1, gf1, bef1, Wf2, bf2, gf2, bef2,
               Wf3, bf3, gf3, bef3, wo, bo, out, acc):
    i = pl.program_id(0)

    @pl.when(i == 0)
    def _():
        acc[:, :] = jnp.full((N_GRAPHS, F_HID), -jnp.inf, _f32)

    dv = dinv[:, :]
    pre = (jnp.dot(a0[:, :] * dv, W2a[:, :], preferred_element_type=_f32, precision=lax.Precision.HIGHEST)
           + jnp.dot(a1[:, :] * dv, W2b[:, :], preferred_element_type=_f32, precision=lax.Precision.HIGHEST)
           + b2[:, :])
    h2 = _ln_relu(pre, g2[:, :], be2[:, :])

    ids = bat[:, :]
    QR = BLK // 4
    for q in range(4):
        ids_q = ids[q * QR:(q + 1) * QR, :]
        h2_q = h2[q * QR:(q + 1) * QR, :]
        lo = bat[q * QR, 0]
        hi = bat[(q + 1) * QR - 1, 0]

        def seg(gid, carry, ids_q=ids_q, h2_q=h2_q):
            mm = jnp.max(jnp.where(ids_q == gid, h2_q, -jnp.inf), axis=0,
                         keepdims=True)
            acc[pl.ds(gid, 1), :] = jnp.maximum(acc[pl.ds(gid, 1), :], mm)
            return carry
        lax.fori_loop(lo, hi + 1, seg, 0)

    @pl.when(i == pl.num_programs(0) - 1)
    def _mlp():
        z = acc[:, :]
        z = _ln_relu(jnp.dot(z, Wf1[:, :], preferred_element_type=_f32, precision=lax.Precision.HIGHEST)
                     + bf1[:, :], gf1[:, :], bef1[:, :])
        z = _ln_relu(jnp.dot(z, Wf2[:, :], preferred_element_type=_f32, precision=lax.Precision.HIGHEST)
                     + bf2[:, :], gf2[:, :], bef2[:, :])
        z = _ln_relu(jnp.dot(z, Wf3[:, :], preferred_element_type=_f32, precision=lax.Precision.HIGHEST)
                     + bf3[:, :], gf3[:, :], bef3[:, :])
        out[:, :] = jnp.sum(z * wo[:, :], axis=1, keepdims=True) + bo[:, :]


def _tc_head(a0, a1, dinv, bat2d, W2a, W2b, b2, g2, be2,
             Wf1, bf1, gf1, bef1, Wf2, bf2, gf2, bef2,
             Wf3, bf3, gf3, bef3, wo, bo):
    half = F_HID // 2
    blk_r = lambda i: (i, 0)
    blk_w = lambda i: (0, 0)
    vec = lambda d: pl.BlockSpec((1, d), blk_w)
    return pl.pallas_call(
        _head_body,
        grid=(GRID,),
        in_specs=[
            pl.BlockSpec((BLK, half), blk_r),
            pl.BlockSpec((BLK, half), blk_r),
            pl.BlockSpec((BLK, 1), blk_r),
            pl.BlockSpec((BLK, 1), blk_r),
            pl.BlockSpec((half, F_HID), blk_w),
            pl.BlockSpec((half, F_HID), blk_w),
            vec(F_HID), vec(F_HID), vec(F_HID),
            pl.BlockSpec((F_HID, F_MLP), blk_w),
            vec(F_MLP), vec(F_MLP), vec(F_MLP),
            pl.BlockSpec((F_MLP, F_MLP), blk_w),
            vec(F_MLP), vec(F_MLP), vec(F_MLP),
            pl.BlockSpec((F_MLP, F_MLP), blk_w),
            vec(F_MLP), vec(F_MLP), vec(F_MLP),
            vec(F_MLP), vec(1),
        ],
        out_specs=pl.BlockSpec((N_GRAPHS, 1), blk_w),
        out_shape=jax.ShapeDtypeStruct((N_GRAPHS, 1), _f32),
        scratch_shapes=[pltpu.VMEM((N_GRAPHS, F_HID), _f32)],
    )(a0, a1, dinv, bat2d, W2a, W2b, b2, g2, be2,
      Wf1, bf1, gf1, bef1, Wf2, bf2, gf2, bef2,
      Wf3, bf3, gf3, bef3, wo, bo)


def kernel(tree_x, edge_index, batch,
           W_g1, b_g1, g_n1, be_n1,
           W_g2, b_g2, g_n2, be_n2,
           W_f1, b_f1, g_f1, be_f1,
           W_f2, b_f2, g_f2, be_f2,
           W_f3, b_f3, g_f3, be_f3,
           W_o, b_o):
    half = F_HID // 2
    src2d = edge_index[0].reshape(ROWS, CHUNK)
    dst2d = edge_index[1].reshape(ROWS, CHUNK)
    src2d2 = edge_index[0].reshape(ROWS2, CHUNK2)
    dst2d2 = edge_index[1].reshape(ROWS2, CHUNK2)

    degp = _sc_deg(dst2d)
    dinv, xs = _tc_prep(degp.T, tree_x)
    aggp = _sc_agg16(src2d, dst2d, xs)
    hs0, hs1 = _tc_layer1(aggp[0], aggp[1], xs, dinv, W_g1,
                          b_g1.reshape(1, F_HID), g_n1.reshape(1, F_HID),
                          be_n1.reshape(1, F_HID))
    ap = _sc_agg256(src2d2, dst2d2, hs0, hs1)
    return _tc_head(ap[0], ap[1], dinv, batch.reshape(N_NODES, 1),
                    W_g2[:half], W_g2[half:],
                    b_g2.reshape(1, F_HID), g_n2.reshape(1, F_HID),
                    be_n2.reshape(1, F_HID),
                    W_f1, b_f1.reshape(1, F_MLP), g_f1.reshape(1, F_MLP),
                    be_f1.reshape(1, F_MLP),
                    W_f2, b_f2.reshape(1, F_MLP), g_f2.reshape(1, F_MLP),
                    be_f2.reshape(1, F_MLP),
                    W_f3, b_f3.reshape(1, F_MLP), g_f3.reshape(1, F_MLP),
                    be_f3.reshape(1, F_MLP),
                    W_o.reshape(1, F_MLP), b_o.reshape(1, 1))
